# baseline (speedup 1.0000x reference)
import functools

import jax
import jax.numpy as jnp
from jax import lax
from jax.experimental import pallas as pl
from jax.experimental.pallas import tpu as pltpu
from jax.experimental.pallas import tpu_sc as plsc

NUM_TOKENS = 16384
HIDDEN = 4096
NUM_EXPERTS = 64
TOP_K = 8
TOKEN_BLOCK = 1024

NUM_CORES = 2
NUM_SUBCORES = 16
NW = NUM_CORES * NUM_SUBCORES
TPW = NUM_TOKENS // NW
LANES = 16
NQ = NUM_EXPERTS // LANES
CHUNK = 256


def _logits_kernel(h_ref, w_ref, logits_ref):
    logits_ref[...] = jax.lax.dot_general(
        h_ref[...], w_ref[...],
        (((1,), (1,)), ((), ())),
        preferred_element_type=jnp.float32,
        precision=jax.lax.Precision.DEFAULT,
    )


def _matmul_logits(hidden_states, weight, tok0, ntok):
    return pl.pallas_call(
        _logits_kernel,
        grid=(ntok // TOKEN_BLOCK,),
        in_specs=[
            pl.BlockSpec((TOKEN_BLOCK, HIDDEN),
                         lambda i: (tok0 // TOKEN_BLOCK + i, 0)),
            pl.BlockSpec((NUM_EXPERTS, HIDDEN), lambda i: (0, 0)),
        ],
        out_specs=pl.BlockSpec((TOKEN_BLOCK, NUM_EXPERTS), lambda i: (i, 0)),
        out_shape=jax.ShapeDtypeStruct((ntok, NUM_EXPERTS), jnp.float32),
    )(hidden_states, weight)


_GDN = lax.GatherDimensionNumbers(
    offset_dims=(), collapsed_slice_dims=(0,), start_index_map=(0,))


def _lane_shuffle(v, perm):
    return lax.gather(v, perm[:, None], _GDN, (1,),
                      mode=lax.GatherScatterMode.PROMISE_IN_BOUNDS)


def _sc_router_body(logits_hbm, merge_hbm, map_hbm, counts_hbm,
                    lbuf, mbuf, ibuf, cbuf):
    ntok = logits_hbm.shape[0]
    tpw = ntok // NW
    chunk = min(CHUNK, tpw)
    wid = lax.axis_index("s") * NUM_CORES + lax.axis_index("c")
    base = wid * tpw

    iota = lax.broadcasted_iota(jnp.int32, (LANES,), 0)
    perms = [iota ^ k for k in (1, 2, 4, 8)]
    qiota = [iota + LANES * q for q in range(NQ)]
    ones_i = jnp.full((LANES,), 1, jnp.int32)
    big_i = jnp.full((LANES,), NUM_EXPERTS, jnp.int32)

    def _butterfly(v, op):
        for perm in perms:
            v = op(v, _lane_shuffle(v, perm))
        return v

    def tok(t, counts):
        l = [lbuf[t, pl.ds(LANES * q, LANES)] for q in range(NQ)]
        e = [jnp.exp(x) for x in l]
        s = _butterfly(e[0] + e[1] + e[2] + e[3], jnp.add)
        p = [x / s for x in e]

        running = list(p)
        selmf = [jnp.zeros((LANES,), jnp.float32) for _ in range(NQ)]
        ksum = jnp.zeros((LANES,), jnp.float32)
        for _ in range(TOP_K):
            v = jnp.maximum(jnp.maximum(running[0], running[1]),
                            jnp.maximum(running[2], running[3]))
            cur = _butterfly(v, jnp.maximum)
            cand = [jnp.where(running[q] == cur, qiota[q], big_i)
                    for q in range(NQ)]
            idx = _butterfly(jnp.minimum(jnp.minimum(cand[0], cand[1]),
                                         jnp.minimum(cand[2], cand[3])),
                             jnp.minimum)
            for q in range(NQ):
                sel = qiota[q] == idx
                selmf[q] = jnp.where(sel, 1.0, selmf[q])
                running[q] = jnp.where(sel, -1.0, running[q])
            ksum = ksum + cur

        new_counts = []
        for q in range(NQ):
            merge = selmf[q] * (p[q] / ksum)
            mbuf[t, pl.ds(LANES * q, LANES)] = merge
            mi = jnp.where(merge > 0.0, ones_i, jnp.zeros((LANES,), jnp.int32))
            ibuf[t, pl.ds(LANES * q, LANES)] = mi
            new_counts.append(counts[q] + mi)
        return tuple(new_counts)

    counts = tuple(jnp.zeros((LANES,), jnp.int32) for _ in range(NQ))
    for c in range(tpw // chunk):
        pltpu.sync_copy(logits_hbm.at[pl.ds(base + c * chunk, chunk)],
                        lbuf.at[pl.ds(0, chunk)])
        counts = lax.fori_loop(0, chunk, tok, counts)
        pltpu.sync_copy(mbuf.at[pl.ds(0, chunk)],
                        merge_hbm.at[pl.ds(base + c * chunk, chunk)])
        pltpu.sync_copy(ibuf.at[pl.ds(0, chunk)],
                        map_hbm.at[pl.ds(base + c * chunk, chunk)])

    for q in range(NQ):
        cbuf[pl.ds(LANES * q, LANES)] = counts[q]
    pltpu.sync_copy(cbuf, counts_hbm.at[wid])


def _sc_router(logits):
    mesh = plsc.VectorSubcoreMesh(core_axis_name="c", subcore_axis_name="s",
                                  num_cores=NUM_CORES,
                                  num_subcores=NUM_SUBCORES)
    return pl.kernel(
        _sc_router_body,
        out_type=[
            jax.ShapeDtypeStruct((logits.shape[0], NUM_EXPERTS), jnp.float32),
            jax.ShapeDtypeStruct((logits.shape[0], NUM_EXPERTS), jnp.int32),
            jax.ShapeDtypeStruct((NW, NUM_EXPERTS), jnp.int32),
        ],
        mesh=mesh,
        scratch_types=[
            pltpu.VMEM((CHUNK, NUM_EXPERTS), jnp.float32),
            pltpu.VMEM((CHUNK, NUM_EXPERTS), jnp.float32),
            pltpu.VMEM((CHUNK, NUM_EXPERTS), jnp.int32),
            pltpu.VMEM((NUM_EXPERTS,), jnp.int32),
        ],
    )(logits)


NUM_SLICES = 4


def kernel(hidden_states, weight):
    ntok = hidden_states.shape[0]
    per = ntok // NUM_SLICES
    logit_slices, merges, maps, counts_list = [], [], [], []
    for s in range(NUM_SLICES):
        logit_slices.append(
            _matmul_logits(hidden_states, weight, s * per, per))
    for s in range(NUM_SLICES):
        m, r, c = _sc_router(logit_slices[s])
        merges.append(m)
        maps.append(r)
        counts_list.append(c)
    merging_probs = jnp.concatenate(merges, axis=0)
    routing_map = jnp.concatenate(maps, axis=0)
    logits = jnp.concatenate(logit_slices, axis=0)
    tokens_per_expert = sum(jnp.sum(c, axis=0) for c in counts_list)
    return merging_probs, routing_map, tokens_per_expert, logits


# speedup vs baseline: 1.5765x; 1.5765x over previous
import functools

import jax
import jax.numpy as jnp
from jax.experimental import pallas as pl

NUM_TOKENS = 16384
HIDDEN = 4096
NUM_EXPERTS = 64
TOP_K = 8
TOKEN_BLOCK = 1024


def _router_kernel(h_ref, w_ref, merge_ref, map_ref, counts_ref,
                   logits_ref):
    i = pl.program_id(0)

    logits = jax.lax.dot_general(
        h_ref[...], w_ref[...],
        (((1,), (1,)), ((), ())),
        preferred_element_type=jnp.float32,
        precision=jax.lax.Precision.DEFAULT,
    )
    logits_ref[...] = logits

    e = jnp.exp(logits)
    probs = e / jnp.sum(e, axis=1, keepdims=True)

    iota = jax.lax.broadcasted_iota(jnp.int32, probs.shape, 1)
    running = probs
    mask = jnp.zeros(probs.shape, dtype=jnp.bool_)
    for _ in range(TOP_K):
        idx = jnp.argmax(running, axis=1, keepdims=True)
        sel = iota == idx
        mask = mask | sel
        running = jnp.where(sel, -1.0, running)

    ksum = jnp.sum(jnp.where(mask, probs, 0.0), axis=1, keepdims=True)
    merge_ref[...] = jnp.where(mask, probs / ksum, 0.0)
    map_i32 = mask.astype(jnp.int32)
    map_ref[...] = map_i32

    part = jnp.sum(map_i32, axis=0)

    @pl.when(i == 0)
    def _init():
        counts_ref[...] = jnp.zeros_like(counts_ref)

    counts_ref[...] += part


@functools.partial(jax.jit, static_argnames=())
def kernel(hidden_states, weight):
    num_tokens = hidden_states.shape[0]
    grid = (num_tokens // TOKEN_BLOCK,)
    out = pl.pallas_call(
        _router_kernel,
        grid=grid,
        in_specs=[
            pl.BlockSpec((TOKEN_BLOCK, HIDDEN), lambda i: (i, 0)),
            pl.BlockSpec((NUM_EXPERTS, HIDDEN), lambda i: (0, 0)),
        ],
        out_specs=[
            pl.BlockSpec((TOKEN_BLOCK, NUM_EXPERTS), lambda i: (i, 0)),
            pl.BlockSpec((TOKEN_BLOCK, NUM_EXPERTS), lambda i: (i, 0)),
            pl.BlockSpec((NUM_EXPERTS,), lambda i: (0,)),
            pl.BlockSpec((TOKEN_BLOCK, NUM_EXPERTS), lambda i: (i, 0)),
        ],
        out_shape=[
            jax.ShapeDtypeStruct((num_tokens, NUM_EXPERTS), jnp.float32),
            jax.ShapeDtypeStruct((num_tokens, NUM_EXPERTS), jnp.int32),
            jax.ShapeDtypeStruct((NUM_EXPERTS,), jnp.int32),
            jax.ShapeDtypeStruct((num_tokens, NUM_EXPERTS), jnp.float32),
        ],
    )(hidden_states, weight)
    merging_probs, routing_map, tokens_per_expert, router_logits = out
    return merging_probs, routing_map, tokens_per_expert, router_logits
